# Initial kernel scaffold; baseline (speedup 1.0000x reference)
import jax
import jax.numpy as jnp
from jax.experimental import pallas as pl


def kernel(positions, params):
    raise NotImplementedError("write your pallas kernel here")


# speedup vs baseline: 6.6527x; 6.6527x over previous
import functools

import numpy as np
import jax
import jax.numpy as jnp
from jax import lax
from jax.experimental import pallas as pl
from jax.experimental.pallas import tpu as pltpu
from jax.experimental.pallas import tpu_sc as plsc


def _sh_np(l, r):
    x, y, z = r[:, 0], r[:, 1], r[:, 2]
    if l == 0:
        return np.ones((r.shape[0], 1))
    if l == 1:
        return np.sqrt(3.0) * np.stack([x, y, z], 1)
    s15 = np.sqrt(15.0)
    return np.stack([s15 * x * y, s15 * y * z,
                     0.5 * np.sqrt(5.0) * (2 * z * z - x * x - y * y),
                     s15 * x * z, 0.5 * s15 * (x * x - y * y)], 1)


def _rot(axis, t):
    c, s = np.cos(t), np.sin(t)
    if axis == 0:
        return np.array([[1., 0., 0.], [0., c, -s], [0., s, c]])
    if axis == 1:
        return np.array([[c, 0., s], [0., 1., 0.], [-s, 0., c]])
    return np.array([[c, -s, 0.], [s, c, 0.], [0., 0., 1.]])


_PTS_LOCAL = None


def _wigner_D(l, R):
    global _PTS_LOCAL
    if _PTS_LOCAL is None:
        rng = np.random.RandomState(0)
        p = rng.randn(512, 3)
        _PTS_LOCAL = p / np.linalg.norm(p, axis=1, keepdims=True)
    A = _sh_np(l, _PTS_LOCAL)
    Bm = _sh_np(l, _PTS_LOCAL @ R.T)
    Dt, _, _, _ = np.linalg.lstsq(A, Bm, rcond=None)
    return Dt.T


def _gens(l):
    eps = 1e-4
    return [(_wigner_D(l, _rot(a, eps)) - _wigner_D(l, _rot(a, -eps))) / (2 * eps)
            for a in range(3)]


def _cg(l1, l2, l3):
    J1, J2, J3 = _gens(l1), _gens(l2), _gens(l3)
    d1, d2, d3 = 2 * l1 + 1, 2 * l2 + 1, 2 * l3 + 1
    rows = []
    for a in range(3):
        K = (np.kron(np.kron(J1[a], np.eye(d2)), np.eye(d3))
             + np.kron(np.kron(np.eye(d1), J2[a]), np.eye(d3))
             + np.kron(np.kron(np.eye(d1), np.eye(d2)), J3[a]))
        rows.append(K)
    M = np.concatenate(rows, 0)
    _, _, Vt = np.linalg.svd(M)
    C = Vt[-1].reshape(d1, d2, d3)
    return (C / np.linalg.norm(C)).astype(np.float32)


_HID = [(32, 0), (16, 1), (8, 2)]
_HID_OFF = [0, 32, 80, 120]
_OUT_OFF = {0: 0, 1: 32, 2: 80}
_KN = 8
_N = 8192
_STRIP = 128
_NSTRIP = _N // _STRIP

_INS = []
for _i, (_mul, _l1) in enumerate(_HID):
    for _l2 in (0, 2):
        for _l3 in range(abs(_l1 - _l2), _l1 + _l2 + 1):
            if _l3 <= 2:
                _INS.append((_i, _l2, _l3, _mul))

_T_L = {0: 40, 1: 40, 2: 64}

_GROUP_M = {}
_GROUP_IDX = {}


def _build_group_constants():
    woff = 0
    t0 = {0: 0, 1: 0, 2: 0}
    M = {l3: np.zeros((6, _T_L[l3], 2 * l3 + 1, 120), np.float32) for l3 in (0, 1, 2)}
    IDX = {l3: np.zeros((_T_L[l3],), np.int32) for l3 in (0, 1, 2)}
    for (i, l2, l3, mul) in _INS:
        l1 = _HID[i][1]
        d1, d2 = 2 * l1 + 1, 2 * l2 + 1
        C = _cg(l1, l2, l3)
        scale = np.sqrt(2 * l3 + 1)
        for u in range(mul):
            IDX[l3][t0[l3] + u] = woff + u
            for jp in range(d2):
                j = 0 if l2 == 0 else 1 + jp
                M[l3][j, t0[l3] + u, :,
                      _HID_OFF[i] + u * d1:_HID_OFF[i] + (u + 1) * d1] += \
                    (C[:, jp, :] * scale).T
        woff += mul
        t0[l3] += mul
    for l3 in (0, 1, 2):
        _GROUP_M[l3] = jnp.asarray(M[l3])
        _GROUP_IDX[l3] = jnp.asarray(IDX[l3])


_build_group_constants()


_MID = 480
_SLAB = 512
_NG = 4


def _build_BT(lp):
    parts = []
    for l3 in (0, 1, 2):
        w = lp["w_tp"][_GROUP_IDX[l3]]
        Bg = jnp.einsum('t,jtkc->jtkc', w, _GROUP_M[l3])
        parts.append(Bg.reshape(6, -1, 120))
    B = jnp.concatenate(parts, axis=1)
    Bp = jnp.pad(B, ((0, 0), (0, _SLAB - _MID), (0, 0)))
    return jnp.transpose(Bp, (2, 0, 1)).reshape(120, 6 * _SLAB)


def _build_mix_weights(lp):
    W0 = lp["W0"].astype(jnp.bfloat16)
    W1e = jnp.kron(lp["W1"], jnp.eye(3, dtype=jnp.float32)).astype(jnp.bfloat16)
    W2e = jnp.kron(lp["W2"], jnp.eye(5, dtype=jnp.float32)).astype(jnp.bfloat16)
    return W0, W1e, W2e


def _knn_phi_body(pos_t_ref, pos_full_ref, pos_blk_ref, dst_ref, phi_ref):
    i = pl.program_id(0)
    pos_t = pos_t_ref[...]
    pos_full = pos_full_ref[...]
    pos_blk = pos_blk_ref[...]
    x2col = jnp.sum(pos_t * pos_t, axis=0, keepdims=True)
    x2row = jnp.sum(pos_blk * pos_blk, axis=1, keepdims=True)
    d = jnp.dot(pos_blk, pos_t, preferred_element_type=jnp.float32)
    v = (x2row + x2col) - 2.0 * d
    col = lax.broadcasted_iota(jnp.int32, v.shape, 1)

    s15 = np.float32(np.sqrt(15.0))
    hq5 = np.float32(0.5 * np.sqrt(5.0))
    dsts = []
    phis = []
    for p in range(_KN + 1):
        m = jnp.min(v, axis=1, keepdims=True)
        is_min = v <= m
        idxk = jnp.min(jnp.where(is_min, col, 2 ** 30), axis=1, keepdims=True)
        onehot = col == idxk
        v = jnp.where(onehot, 1e30, v)
        if p == 0:
            continue
        posk = jnp.dot(onehot.astype(jnp.float32), pos_full,
                       preferred_element_type=jnp.float32,
                       precision=lax.Precision.HIGHEST)
        dsts.append(idxk)
        r = posk - pos_blk
        n = jnp.sqrt(jnp.sum(r * r, axis=1, keepdims=True))
        u = r / jnp.maximum(n, 1e-12)
        ux, uy, uz = u[:, 0:1], u[:, 1:2], u[:, 2:3]
        phis.append(jnp.concatenate([
            jnp.ones_like(ux),
            s15 * ux * uy,
            s15 * uy * uz,
            hq5 * (2.0 * uz * uz - ux * ux - uy * uy),
            s15 * ux * uz,
            0.5 * s15 * (ux * ux - uy * uy),
        ], axis=1))
    dst_ref[...] = jnp.concatenate(dsts, axis=1)
    phi_ref[...] = jnp.concatenate(phis, axis=1)


def _knn_phi_call(pos, pos_t):
    return pl.pallas_call(
        _knn_phi_body,
        grid=(_NSTRIP,),
        in_specs=[
            pl.BlockSpec((3, _N), lambda i: (0, 0)),
            pl.BlockSpec((_N, 3), lambda i: (0, 0)),
            pl.BlockSpec((_STRIP, 3), lambda i: (i, 0)),
        ],
        out_specs=[
            pl.BlockSpec((_STRIP, _KN), lambda i: (i, 0)),
            pl.BlockSpec((_STRIP, 6 * _KN), lambda i: (i, 0)),
        ],
        out_shape=[
            jax.ShapeDtypeStruct((_N, _KN), jnp.int32),
            jax.ShapeDtypeStruct((_N, 6 * _KN), jnp.float32),
        ],
    )(pos_t, pos, pos)


def _combine_msg(xcur, BT_ref, phi):
    Y = jnp.dot(xcur, BT_ref[...], preferred_element_type=jnp.float32,
                precision=lax.Precision.HIGHEST)
    outs = [[] for _ in range(_NG)]
    for k in range(_KN):
        acc = phi[:, 6 * k:6 * k + 1] * Y[:, 0:_SLAB]
        for j in range(1, 6):
            acc = acc + phi[:, 6 * k + j:6 * k + j + 1] * Y[:, _SLAB * j:_SLAB * (j + 1)]
        for g in range(_NG):
            outs[g].append(acc[:, g * 128:(g + 1) * 128])
    return [jnp.concatenate(o, axis=1) for o in outs]


def _mix_and_stats(a0, a1, a2, a3, W0_ref, W1e_ref, W2e_ref):
    g0 = a0[:, 0:40].astype(jnp.bfloat16)
    g1 = jnp.concatenate([a0[:, 40:128], a1[:, 0:32]], axis=1).astype(jnp.bfloat16)
    g2 = jnp.concatenate([a1[:, 32:128], a2, a3[:, 0:96]], axis=1).astype(jnp.bfloat16)
    y0 = jnp.dot(g0, W0_ref[...], preferred_element_type=jnp.float32) / np.float32(np.sqrt(40.0))
    y1 = jnp.dot(g1, W1e_ref[...], preferred_element_type=jnp.float32) / np.float32(np.sqrt(40.0))
    y2 = jnp.dot(g2, W2e_ref[...], preferred_element_type=jnp.float32) / np.float32(np.sqrt(64.0))
    y = jnp.concatenate([y0, y1, y2], axis=1)
    s0 = jnp.sum(y0, axis=0, keepdims=True)
    ss0 = jnp.sum(y0 * y0, axis=0, keepdims=True)
    ss1 = jnp.sum(y1 * y1, axis=0, keepdims=True)
    ss2 = jnp.sum(y2 * y2, axis=0, keepdims=True)
    z = jnp.zeros((1, 104), jnp.float32)
    st = jnp.concatenate([s0, ss0, ss1, ss2, z], axis=1)[:, None, :]
    return y, st


def _apply_bn(y, xprev, stats_ref, bnw_ref, bnb_ref):
    st = jnp.sum(stats_ref[...][:, 0, :], axis=0, keepdims=True)
    ninv = np.float32(1.0 / _N)
    mean0 = st[:, 0:32] * ninv
    var0 = st[:, 32:64] * ninv - mean0 * mean0
    bnw = bnw_ref[...]
    scale0 = bnw[:, 0:32] / jnp.sqrt(var0 + 1e-5)
    bias0 = bnb_ref[...] - mean0 * scale0
    y0 = y[:, 0:32] * scale0 + bias0
    ss1 = st[:, 64:112] * np.float32(1.0 / (_N * 3))
    g1r = lax.broadcasted_iota(jnp.int32, (48, 48), 0) // 3
    g1c = lax.broadcasted_iota(jnp.int32, (48, 48), 1) // 3
    n2_1 = jnp.dot(ss1, (g1r == g1c).astype(jnp.float32),
                   preferred_element_type=jnp.float32,
                   precision=lax.Precision.HIGHEST)
    scale1 = bnw[:, 32:80] / jnp.sqrt(n2_1 + 1e-5)
    y1 = y[:, 32:80] * scale1
    ss2 = st[:, 112:152] * np.float32(1.0 / (_N * 5))
    g2r = lax.broadcasted_iota(jnp.int32, (40, 40), 0) // 5
    g2c = lax.broadcasted_iota(jnp.int32, (40, 40), 1) // 5
    n2_2 = jnp.dot(ss2, (g2r == g2c).astype(jnp.float32),
                   preferred_element_type=jnp.float32,
                   precision=lax.Precision.HIGHEST)
    scale2 = bnw[:, 80:120] / jnp.sqrt(n2_2 + 1e-5)
    y2 = y[:, 80:120] * scale2
    xn = jnp.concatenate([y0, y1, y2], axis=1)
    return jax.nn.relu(xn) + xprev


_MSG_SPEC = pl.BlockSpec((_NG, _STRIP, _KN * 128), lambda i: (0, i, 0))
_MSG_SHAPE = jax.ShapeDtypeStruct((_NG, _N, _KN * 128), jnp.float32)


def _write_msg(msg_ref, msgs):
    for g in range(_NG):
        msg_ref[g] = msgs[g]


def _combine0_body(x_ref, BT_ref, phi_ref, msg_ref):
    _write_msg(msg_ref, _combine_msg(x_ref[...], BT_ref, phi_ref[...]))


def _combine0_call(x, BT, phi):
    return pl.pallas_call(
        _combine0_body,
        grid=(_NSTRIP,),
        in_specs=[
            pl.BlockSpec((_STRIP, 120), lambda i: (i, 0)),
            pl.BlockSpec((120, 6 * _SLAB), lambda i: (0, 0)),
            pl.BlockSpec((_STRIP, 48), lambda i: (i, 0)),
        ],
        out_specs=_MSG_SPEC,
        out_shape=_MSG_SHAPE,
    )(x, BT, phi)


def _mix_stats_body(acc_ref, W0_ref, W1e_ref, W2e_ref, y_ref, st_ref):
    y, st = _mix_and_stats(acc_ref[0], acc_ref[1], acc_ref[2], acc_ref[3],
                           W0_ref, W1e_ref, W2e_ref)
    y_ref[...] = y
    st_ref[...] = st


def _mix_stats_call(acc, W0, W1e, W2e):
    return pl.pallas_call(
        _mix_stats_body,
        grid=(_NSTRIP,),
        in_specs=[
            pl.BlockSpec((_NG, _STRIP, 128), lambda i: (0, i, 0)),
            pl.BlockSpec((40, 32), lambda i: (0, 0)),
            pl.BlockSpec((120, 48), lambda i: (0, 0)),
            pl.BlockSpec((320, 40), lambda i: (0, 0)),
        ],
        out_specs=[
            pl.BlockSpec((_STRIP, 120), lambda i: (i, 0)),
            pl.BlockSpec((1, 1, 256), lambda i: (i, 0, 0)),
        ],
        out_shape=[
            jax.ShapeDtypeStruct((_N, 120), jnp.float32),
            jax.ShapeDtypeStruct((_NSTRIP, 1, 256), jnp.float32),
        ],
    )(acc, W0, W1e, W2e)


def _bn_combine_body(y_ref, x_ref, stats_ref, bnw_ref, bnb_ref,
                     BT_ref, phi_ref, xout_ref, msg_ref):
    xnew = _apply_bn(y_ref[...], x_ref[...], stats_ref, bnw_ref, bnb_ref)
    xout_ref[...] = xnew
    _write_msg(msg_ref, _combine_msg(xnew, BT_ref, phi_ref[...]))


def _bn_combine_call(y, x, stats, bnw, bnb, BT, phi):
    return pl.pallas_call(
        _bn_combine_body,
        grid=(_NSTRIP,),
        in_specs=[
            pl.BlockSpec((_STRIP, 120), lambda i: (i, 0)),
            pl.BlockSpec((_STRIP, 120), lambda i: (i, 0)),
            pl.BlockSpec((_NSTRIP, 1, 256), lambda i: (0, 0, 0)),
            pl.BlockSpec((1, 120), lambda i: (0, 0)),
            pl.BlockSpec((1, 32), lambda i: (0, 0)),
            pl.BlockSpec((120, 6 * _SLAB), lambda i: (0, 0)),
            pl.BlockSpec((_STRIP, 48), lambda i: (i, 0)),
        ],
        out_specs=[
            pl.BlockSpec((_STRIP, 120), lambda i: (i, 0)),
            _MSG_SPEC,
        ],
        out_shape=[
            jax.ShapeDtypeStruct((_N, 120), jnp.float32),
            _MSG_SHAPE,
        ],
    )(y, x, stats, bnw, bnb, BT, phi)


def _bn_head_body(y_ref, x_ref, stats_ref, bnw_ref, bnb_ref,
                  Wf_ref, A1_ref, b1_ref, A2_ref, b2_ref, out_ref):
    xnew = _apply_bn(y_ref[...], x_ref[...], stats_ref, bnw_ref, bnb_ref)
    h = jnp.dot(xnew[:, :32].astype(jnp.bfloat16), Wf_ref[...],
                preferred_element_type=jnp.float32) / np.float32(np.sqrt(32.0))
    h = jax.nn.relu(h)
    h = jax.nn.relu(jnp.dot(h.astype(jnp.bfloat16), A1_ref[...],
                            preferred_element_type=jnp.float32) + b1_ref[...])
    out_ref[...] = (jnp.dot(h.astype(jnp.bfloat16), A2_ref[...],
                            preferred_element_type=jnp.float32) + b2_ref[...])


def _bn_head_call(y, x, stats, bnw, bnb, Wf, A1, b1, A2, b2):
    return pl.pallas_call(
        _bn_head_body,
        grid=(_NSTRIP,),
        in_specs=[
            pl.BlockSpec((_STRIP, 120), lambda i: (i, 0)),
            pl.BlockSpec((_STRIP, 120), lambda i: (i, 0)),
            pl.BlockSpec((_NSTRIP, 1, 256), lambda i: (0, 0, 0)),
            pl.BlockSpec((1, 120), lambda i: (0, 0)),
            pl.BlockSpec((1, 32), lambda i: (0, 0)),
            pl.BlockSpec((32, 32), lambda i: (0, 0)),
            pl.BlockSpec((32, 16), lambda i: (0, 0)),
            pl.BlockSpec((1, 16), lambda i: (0, 0)),
            pl.BlockSpec((16, 8), lambda i: (0, 0)),
            pl.BlockSpec((1, 8), lambda i: (0, 0)),
        ],
        out_specs=pl.BlockSpec((_STRIP, 8), lambda i: (i, 0)),
        out_shape=jax.ShapeDtypeStruct((_N, 8), jnp.float32),
    )(y, x, stats, bnw, bnb, Wf, A1, b1, A2, b2)


_ROWS_PER_TILE = _N // 16
_CHUNKS_PER_TILE = 32


def _scatter_body(msg_hbm, dstr_hbm, zeros_hbm, out_hbm, idx_v, msg_v, acc_sh):
    cid = lax.axis_index("c")
    sid = lax.axis_index("s")
    base_row = sid * _CHUNKS_PER_TILE
    pltpu.sync_copy(dstr_hbm.at[pl.ds(base_row, _CHUNKS_PER_TILE)], idx_v)
    for gg in range(2):
        g = cid * 2 + gg
        pltpu.sync_copy(zeros_hbm.at[pl.ds(sid * _ROWS_PER_TILE, _ROWS_PER_TILE)],
                        acc_sh.at[pl.ds(sid * _ROWS_PER_TILE, _ROWS_PER_TILE)])
        plsc.subcore_barrier()
        for j in range(_CHUNKS_PER_TILE):
            pltpu.sync_copy(msg_hbm.at[g, pl.ds((base_row + j) * 128, 128)], msg_v)
            pltpu.sync_copy(msg_v, acc_sh.at[idx_v.at[j]], add=True)
        plsc.subcore_barrier()
        pltpu.sync_copy(acc_sh.at[pl.ds(sid * _ROWS_PER_TILE, _ROWS_PER_TILE)],
                        out_hbm.at[g, pl.ds(sid * _ROWS_PER_TILE, _ROWS_PER_TILE)])


@functools.lru_cache(maxsize=1)
def _get_sc_scatter():
    mesh = plsc.VectorSubcoreMesh(core_axis_name="c", subcore_axis_name="s")
    return pl.kernel(
        _scatter_body,
        out_type=jax.ShapeDtypeStruct((_NG, _N, 128), jnp.float32),
        mesh=mesh,
        scratch_types=[
            pltpu.VMEM((_CHUNKS_PER_TILE, 128), jnp.int32),
            pltpu.VMEM((128, 128), jnp.float32),
            pltpu.VMEM_SHARED((_N, 128), jnp.float32),
        ],
    )


def _scatter(msg, dstr, zeros):
    return _get_sc_scatter()(msg, dstr, zeros)


def kernel(positions, params):
    B, P, _ = positions.shape
    pos = positions.reshape(-1, 3)
    dst, phi = _knn_phi_call(pos, pos.T)
    dstr = dst.reshape(512, 128)
    zeros = jnp.zeros((_N, 128), jnp.float32)

    x = jnp.concatenate([
        jnp.broadcast_to(params["w_embed"], (_N, 32)),
        jnp.zeros((_N, _HID_OFF[-1] - 32), jnp.float32)], axis=1)

    layers = params["layers"]
    BTs = [_build_BT(lp) for lp in layers]
    mixws = [_build_mix_weights(lp) for lp in layers]
    bnw_e = [jnp.concatenate([
        lp["bn_w"][0:32],
        jnp.repeat(lp["bn_w"][32:48], 3),
        jnp.repeat(lp["bn_w"][48:56], 5)]).reshape(1, 120) for lp in layers]
    bnb = [lp["bn_b"].reshape(1, 32) for lp in layers]

    msg = _combine0_call(x, BTs[0], phi)
    for li in range(3):
        acc = _scatter(msg.reshape(_NG, _N * _KN, 128), dstr, zeros)
        y, stats = _mix_stats_call(acc, *mixws[li])
        if li < 2:
            x, msg = _bn_combine_call(y, x, stats, bnw_e[li], bnb[li],
                                      BTs[li + 1], phi)
        else:
            A2p = jnp.pad(params["A2"], ((0, 0), (0, 6))).astype(jnp.bfloat16)
            b2p = jnp.pad(params["b2"], (0, 6)).reshape(1, 8)
            out = _bn_head_call(y, x, stats, bnw_e[li], bnb[li],
                                params["Wf"].astype(jnp.bfloat16),
                                params["A1"].astype(jnp.bfloat16),
                                params["b1"].reshape(1, 16), A2p, b2p)
    return out[:, :2].reshape(B, P, 2)


# speedup vs baseline: 6.8165x; 1.0246x over previous
import functools

import numpy as np
import jax
import jax.numpy as jnp
from jax import lax
from jax.experimental import pallas as pl
from jax.experimental.pallas import tpu as pltpu
from jax.experimental.pallas import tpu_sc as plsc


def _sh_np(l, r):
    x, y, z = r[:, 0], r[:, 1], r[:, 2]
    if l == 0:
        return np.ones((r.shape[0], 1))
    if l == 1:
        return np.sqrt(3.0) * np.stack([x, y, z], 1)
    s15 = np.sqrt(15.0)
    return np.stack([s15 * x * y, s15 * y * z,
                     0.5 * np.sqrt(5.0) * (2 * z * z - x * x - y * y),
                     s15 * x * z, 0.5 * s15 * (x * x - y * y)], 1)


def _rot(axis, t):
    c, s = np.cos(t), np.sin(t)
    if axis == 0:
        return np.array([[1., 0., 0.], [0., c, -s], [0., s, c]])
    if axis == 1:
        return np.array([[c, 0., s], [0., 1., 0.], [-s, 0., c]])
    return np.array([[c, -s, 0.], [s, c, 0.], [0., 0., 1.]])


_PTS_LOCAL = None


def _wigner_D(l, R):
    global _PTS_LOCAL
    if _PTS_LOCAL is None:
        rng = np.random.RandomState(0)
        p = rng.randn(512, 3)
        _PTS_LOCAL = p / np.linalg.norm(p, axis=1, keepdims=True)
    A = _sh_np(l, _PTS_LOCAL)
    Bm = _sh_np(l, _PTS_LOCAL @ R.T)
    Dt, _, _, _ = np.linalg.lstsq(A, Bm, rcond=None)
    return Dt.T


def _gens(l):
    eps = 1e-4
    return [(_wigner_D(l, _rot(a, eps)) - _wigner_D(l, _rot(a, -eps))) / (2 * eps)
            for a in range(3)]


def _cg(l1, l2, l3):
    J1, J2, J3 = _gens(l1), _gens(l2), _gens(l3)
    d1, d2, d3 = 2 * l1 + 1, 2 * l2 + 1, 2 * l3 + 1
    rows = []
    for a in range(3):
        K = (np.kron(np.kron(J1[a], np.eye(d2)), np.eye(d3))
             + np.kron(np.kron(np.eye(d1), J2[a]), np.eye(d3))
             + np.kron(np.kron(np.eye(d1), np.eye(d2)), J3[a]))
        rows.append(K)
    M = np.concatenate(rows, 0)
    _, _, Vt = np.linalg.svd(M)
    C = Vt[-1].reshape(d1, d2, d3)
    return (C / np.linalg.norm(C)).astype(np.float32)


_HID = [(32, 0), (16, 1), (8, 2)]
_HID_OFF = [0, 32, 80, 120]
_OUT_OFF = {0: 0, 1: 32, 2: 80}
_KN = 8
_N = 8192
_STRIP = 128
_NSTRIP = _N // _STRIP

_INS = []
for _i, (_mul, _l1) in enumerate(_HID):
    for _l2 in (0, 2):
        for _l3 in range(abs(_l1 - _l2), _l1 + _l2 + 1):
            if _l3 <= 2:
                _INS.append((_i, _l2, _l3, _mul))

_T_L = {0: 40, 1: 40, 2: 64}

_GROUP_M = {}
_GROUP_IDX = {}


def _build_group_constants():
    woff = 0
    t0 = {0: 0, 1: 0, 2: 0}
    M = {l3: np.zeros((6, _T_L[l3], 2 * l3 + 1, 120), np.float32) for l3 in (0, 1, 2)}
    IDX = {l3: np.zeros((_T_L[l3],), np.int32) for l3 in (0, 1, 2)}
    for (i, l2, l3, mul) in _INS:
        l1 = _HID[i][1]
        d1, d2 = 2 * l1 + 1, 2 * l2 + 1
        C = _cg(l1, l2, l3)
        scale = np.sqrt(2 * l3 + 1)
        for u in range(mul):
            IDX[l3][t0[l3] + u] = woff + u
            for jp in range(d2):
                j = 0 if l2 == 0 else 1 + jp
                M[l3][j, t0[l3] + u, :,
                      _HID_OFF[i] + u * d1:_HID_OFF[i] + (u + 1) * d1] += \
                    (C[:, jp, :] * scale).T
        woff += mul
        t0[l3] += mul
    for l3 in (0, 1, 2):
        _GROUP_M[l3] = jnp.asarray(M[l3])
        _GROUP_IDX[l3] = jnp.asarray(IDX[l3])


_build_group_constants()


_MID = 480
_SLAB = 512
_NG = 4


def _build_BT(lp):
    parts = []
    for l3 in (0, 1, 2):
        w = lp["w_tp"][_GROUP_IDX[l3]]
        Bg = jnp.einsum('t,jtkc->jtkc', w, _GROUP_M[l3])
        parts.append(Bg.reshape(6, -1, 120))
    B = jnp.concatenate(parts, axis=1)
    Bp = jnp.pad(B, ((0, 0), (0, _SLAB - _MID), (0, 0)))
    return jnp.transpose(Bp, (2, 0, 1)).reshape(120, 6 * _SLAB)


def _build_mix_weights(lp):
    W0 = lp["W0"].astype(jnp.bfloat16)
    W1e = jnp.kron(lp["W1"], jnp.eye(3, dtype=jnp.float32)).astype(jnp.bfloat16)
    W2e = jnp.kron(lp["W2"], jnp.eye(5, dtype=jnp.float32)).astype(jnp.bfloat16)
    return W0, W1e, W2e


def _knn_phi_body(pos_t_ref, pos_full_ref, pos_blk_ref, dst_ref, phi_ref):
    i = pl.program_id(0)
    pos_t = pos_t_ref[...]
    pos_full = pos_full_ref[...]
    pos_blk = pos_blk_ref[...]
    x2col = jnp.sum(pos_t * pos_t, axis=0, keepdims=True)
    x2row = jnp.sum(pos_blk * pos_blk, axis=1, keepdims=True)
    d = jnp.dot(pos_blk, pos_t, preferred_element_type=jnp.float32)
    v = (x2row + x2col) - 2.0 * d
    col = lax.broadcasted_iota(jnp.int32, v.shape, 1)

    s15 = np.float32(np.sqrt(15.0))
    hq5 = np.float32(0.5 * np.sqrt(5.0))
    dsts = []
    phis = []
    for p in range(_KN + 1):
        m = jnp.min(v, axis=1, keepdims=True)
        is_min = v <= m
        idxk = jnp.min(jnp.where(is_min, col, 2 ** 30), axis=1, keepdims=True)
        onehot = col == idxk
        v = jnp.where(onehot, 1e30, v)
        if p == 0:
            continue
        posk = jnp.dot(onehot.astype(jnp.float32), pos_full,
                       preferred_element_type=jnp.float32,
                       precision=lax.Precision.HIGHEST)
        dsts.append(idxk)
        r = posk - pos_blk
        n = jnp.sqrt(jnp.sum(r * r, axis=1, keepdims=True))
        u = r / jnp.maximum(n, 1e-12)
        ux, uy, uz = u[:, 0:1], u[:, 1:2], u[:, 2:3]
        phis.append(jnp.concatenate([
            jnp.ones_like(ux),
            s15 * ux * uy,
            s15 * uy * uz,
            hq5 * (2.0 * uz * uz - ux * ux - uy * uy),
            s15 * ux * uz,
            0.5 * s15 * (ux * ux - uy * uy),
        ], axis=1))
    dst_ref[...] = jnp.concatenate(dsts, axis=1)
    phi_ref[...] = jnp.concatenate(phis, axis=1)


def _knn_phi_call(pos, pos_t):
    return pl.pallas_call(
        _knn_phi_body,
        grid=(_NSTRIP,),
        in_specs=[
            pl.BlockSpec((3, _N), lambda i: (0, 0)),
            pl.BlockSpec((_N, 3), lambda i: (0, 0)),
            pl.BlockSpec((_STRIP, 3), lambda i: (i, 0)),
        ],
        out_specs=[
            pl.BlockSpec((_STRIP, _KN), lambda i: (i, 0)),
            pl.BlockSpec((_STRIP, 6 * _KN), lambda i: (i, 0)),
        ],
        out_shape=[
            jax.ShapeDtypeStruct((_N, _KN), jnp.int32),
            jax.ShapeDtypeStruct((_N, 6 * _KN), jnp.float32),
        ],
    )(pos_t, pos, pos)


def _combine_msg(xcur, BT_ref, phi):
    Y = jnp.dot(xcur, BT_ref[...], preferred_element_type=jnp.float32,
                precision=lax.Precision.HIGHEST)
    outs = [[] for _ in range(_NG)]
    for k in range(_KN):
        acc = phi[:, 6 * k:6 * k + 1] * Y[:, 0:_SLAB]
        for j in range(1, 6):
            acc = acc + phi[:, 6 * k + j:6 * k + j + 1] * Y[:, _SLAB * j:_SLAB * (j + 1)]
        for g in range(_NG):
            outs[g].append(acc[:, g * 128:(g + 1) * 128])
    return [jnp.concatenate(o, axis=1) for o in outs]


def _mix_and_stats(a0, a1, a2, a3, W0_ref, W1e_ref, W2e_ref):
    g0 = a0[:, 0:40].astype(jnp.bfloat16)
    g1 = jnp.concatenate([a0[:, 40:128], a1[:, 0:32]], axis=1).astype(jnp.bfloat16)
    g2 = jnp.concatenate([a1[:, 32:128], a2, a3[:, 0:96]], axis=1).astype(jnp.bfloat16)
    y0 = jnp.dot(g0, W0_ref[...], preferred_element_type=jnp.float32) / np.float32(np.sqrt(40.0))
    y1 = jnp.dot(g1, W1e_ref[...], preferred_element_type=jnp.float32) / np.float32(np.sqrt(40.0))
    y2 = jnp.dot(g2, W2e_ref[...], preferred_element_type=jnp.float32) / np.float32(np.sqrt(64.0))
    y = jnp.concatenate([y0, y1, y2], axis=1)
    s0 = jnp.sum(y0, axis=0, keepdims=True)
    ss0 = jnp.sum(y0 * y0, axis=0, keepdims=True)
    ss1 = jnp.sum(y1 * y1, axis=0, keepdims=True)
    ss2 = jnp.sum(y2 * y2, axis=0, keepdims=True)
    z = jnp.zeros((1, 104), jnp.float32)
    st = jnp.concatenate([s0, ss0, ss1, ss2, z], axis=1)[:, None, :]
    return y, st


def _apply_bn(y, xprev, stats_ref, bnw_ref, bnb_ref):
    st = jnp.sum(stats_ref[...][:, 0, :], axis=0, keepdims=True)
    ninv = np.float32(1.0 / _N)
    mean0 = st[:, 0:32] * ninv
    var0 = st[:, 32:64] * ninv - mean0 * mean0
    bnw = bnw_ref[...]
    scale0 = bnw[:, 0:32] / jnp.sqrt(var0 + 1e-5)
    bias0 = bnb_ref[...] - mean0 * scale0
    y0 = y[:, 0:32] * scale0 + bias0
    ss1 = st[:, 64:112] * np.float32(1.0 / (_N * 3))
    g1r = lax.broadcasted_iota(jnp.int32, (48, 48), 0) // 3
    g1c = lax.broadcasted_iota(jnp.int32, (48, 48), 1) // 3
    n2_1 = jnp.dot(ss1, (g1r == g1c).astype(jnp.float32),
                   preferred_element_type=jnp.float32,
                   precision=lax.Precision.HIGHEST)
    scale1 = bnw[:, 32:80] / jnp.sqrt(n2_1 + 1e-5)
    y1 = y[:, 32:80] * scale1
    ss2 = st[:, 112:152] * np.float32(1.0 / (_N * 5))
    g2r = lax.broadcasted_iota(jnp.int32, (40, 40), 0) // 5
    g2c = lax.broadcasted_iota(jnp.int32, (40, 40), 1) // 5
    n2_2 = jnp.dot(ss2, (g2r == g2c).astype(jnp.float32),
                   preferred_element_type=jnp.float32,
                   precision=lax.Precision.HIGHEST)
    scale2 = bnw[:, 80:120] / jnp.sqrt(n2_2 + 1e-5)
    y2 = y[:, 80:120] * scale2
    xn = jnp.concatenate([y0, y1, y2], axis=1)
    return jax.nn.relu(xn) + xprev


_MSG_SPEC = pl.BlockSpec((_NG, _STRIP, _KN * 128), lambda i: (0, i, 0))
_MSG_SHAPE = jax.ShapeDtypeStruct((_NG, _N, _KN * 128), jnp.float32)


def _write_msg(msg_ref, msgs):
    for g in range(_NG):
        msg_ref[g] = msgs[g]


def _combine0_body(x_ref, BT_ref, phi_ref, msg_ref):
    _write_msg(msg_ref, _combine_msg(x_ref[...], BT_ref, phi_ref[...]))


def _combine0_call(x, BT, phi):
    return pl.pallas_call(
        _combine0_body,
        grid=(_NSTRIP,),
        in_specs=[
            pl.BlockSpec((_STRIP, 120), lambda i: (i, 0)),
            pl.BlockSpec((120, 6 * _SLAB), lambda i: (0, 0)),
            pl.BlockSpec((_STRIP, 48), lambda i: (i, 0)),
        ],
        out_specs=_MSG_SPEC,
        out_shape=_MSG_SHAPE,
    )(x, BT, phi)


def _mix_stats_body(acc_ref, W0_ref, W1e_ref, W2e_ref, y_ref, st_ref):
    y, st = _mix_and_stats(acc_ref[0], acc_ref[1], acc_ref[2], acc_ref[3],
                           W0_ref, W1e_ref, W2e_ref)
    y_ref[...] = y
    st_ref[...] = st


def _mix_stats_call(acc, W0, W1e, W2e):
    return pl.pallas_call(
        _mix_stats_body,
        grid=(_NSTRIP,),
        in_specs=[
            pl.BlockSpec((_NG, _STRIP, 128), lambda i: (0, i, 0)),
            pl.BlockSpec((40, 32), lambda i: (0, 0)),
            pl.BlockSpec((120, 48), lambda i: (0, 0)),
            pl.BlockSpec((320, 40), lambda i: (0, 0)),
        ],
        out_specs=[
            pl.BlockSpec((_STRIP, 120), lambda i: (i, 0)),
            pl.BlockSpec((1, 1, 256), lambda i: (i, 0, 0)),
        ],
        out_shape=[
            jax.ShapeDtypeStruct((_N, 120), jnp.float32),
            jax.ShapeDtypeStruct((_NSTRIP, 1, 256), jnp.float32),
        ],
    )(acc, W0, W1e, W2e)


def _bn_combine_body(y_ref, x_ref, stats_ref, bnw_ref, bnb_ref,
                     BT_ref, phi_ref, xout_ref, msg_ref):
    xnew = _apply_bn(y_ref[...], x_ref[...], stats_ref, bnw_ref, bnb_ref)
    xout_ref[...] = xnew
    _write_msg(msg_ref, _combine_msg(xnew, BT_ref, phi_ref[...]))


def _bn_combine_call(y, x, stats, bnw, bnb, BT, phi):
    return pl.pallas_call(
        _bn_combine_body,
        grid=(_NSTRIP,),
        in_specs=[
            pl.BlockSpec((_STRIP, 120), lambda i: (i, 0)),
            pl.BlockSpec((_STRIP, 120), lambda i: (i, 0)),
            pl.BlockSpec((_NSTRIP, 1, 256), lambda i: (0, 0, 0)),
            pl.BlockSpec((1, 120), lambda i: (0, 0)),
            pl.BlockSpec((1, 32), lambda i: (0, 0)),
            pl.BlockSpec((120, 6 * _SLAB), lambda i: (0, 0)),
            pl.BlockSpec((_STRIP, 48), lambda i: (i, 0)),
        ],
        out_specs=[
            pl.BlockSpec((_STRIP, 120), lambda i: (i, 0)),
            _MSG_SPEC,
        ],
        out_shape=[
            jax.ShapeDtypeStruct((_N, 120), jnp.float32),
            _MSG_SHAPE,
        ],
    )(y, x, stats, bnw, bnb, BT, phi)


def _bn_head_body(y_ref, x_ref, stats_ref, bnw_ref, bnb_ref,
                  Wf_ref, A1_ref, b1_ref, A2_ref, b2_ref, out_ref):
    xnew = _apply_bn(y_ref[...], x_ref[...], stats_ref, bnw_ref, bnb_ref)
    h = jnp.dot(xnew[:, :32].astype(jnp.bfloat16), Wf_ref[...],
                preferred_element_type=jnp.float32) / np.float32(np.sqrt(32.0))
    h = jax.nn.relu(h)
    h = jax.nn.relu(jnp.dot(h.astype(jnp.bfloat16), A1_ref[...],
                            preferred_element_type=jnp.float32) + b1_ref[...])
    out_ref[...] = (jnp.dot(h.astype(jnp.bfloat16), A2_ref[...],
                            preferred_element_type=jnp.float32) + b2_ref[...])


def _bn_head_call(y, x, stats, bnw, bnb, Wf, A1, b1, A2, b2):
    return pl.pallas_call(
        _bn_head_body,
        grid=(_NSTRIP,),
        in_specs=[
            pl.BlockSpec((_STRIP, 120), lambda i: (i, 0)),
            pl.BlockSpec((_STRIP, 120), lambda i: (i, 0)),
            pl.BlockSpec((_NSTRIP, 1, 256), lambda i: (0, 0, 0)),
            pl.BlockSpec((1, 120), lambda i: (0, 0)),
            pl.BlockSpec((1, 32), lambda i: (0, 0)),
            pl.BlockSpec((32, 32), lambda i: (0, 0)),
            pl.BlockSpec((32, 16), lambda i: (0, 0)),
            pl.BlockSpec((1, 16), lambda i: (0, 0)),
            pl.BlockSpec((16, 8), lambda i: (0, 0)),
            pl.BlockSpec((1, 8), lambda i: (0, 0)),
        ],
        out_specs=pl.BlockSpec((_STRIP, 8), lambda i: (i, 0)),
        out_shape=jax.ShapeDtypeStruct((_N, 8), jnp.float32),
    )(y, x, stats, bnw, bnb, Wf, A1, b1, A2, b2)


_ROWS_PER_TILE = _N // 16
_CHUNKS_PER_TILE = 32


def _scatter_body(msg_hbm, dstr_hbm, zeros_hbm, out_hbm, idx_v, msg_v0, msg_v1,
                  sem, acc_sh):
    cid = lax.axis_index("c")
    sid = lax.axis_index("s")
    base_row = sid * _CHUNKS_PER_TILE
    bufs = (msg_v0, msg_v1)
    pltpu.sync_copy(dstr_hbm.at[pl.ds(base_row, _CHUNKS_PER_TILE)], idx_v)
    for gg in range(2):
        g = cid * 2 + gg
        pltpu.sync_copy(zeros_hbm.at[pl.ds(sid * _ROWS_PER_TILE, _ROWS_PER_TILE)],
                        acc_sh.at[pl.ds(sid * _ROWS_PER_TILE, _ROWS_PER_TILE)])
        plsc.subcore_barrier()
        h = pltpu.async_copy(msg_hbm.at[g, pl.ds(base_row * 128, 128)], bufs[0], sem)
        for j in range(_CHUNKS_PER_TILE):
            h.wait()
            if j + 1 < _CHUNKS_PER_TILE:
                h = pltpu.async_copy(
                    msg_hbm.at[g, pl.ds((base_row + j + 1) * 128, 128)],
                    bufs[(j + 1) % 2], sem)
            pltpu.sync_copy(bufs[j % 2], acc_sh.at[idx_v.at[j]], add=True)
        plsc.subcore_barrier()
        pltpu.sync_copy(acc_sh.at[pl.ds(sid * _ROWS_PER_TILE, _ROWS_PER_TILE)],
                        out_hbm.at[g, pl.ds(sid * _ROWS_PER_TILE, _ROWS_PER_TILE)])


@functools.lru_cache(maxsize=1)
def _get_sc_scatter():
    mesh = plsc.VectorSubcoreMesh(core_axis_name="c", subcore_axis_name="s")
    return pl.kernel(
        _scatter_body,
        out_type=jax.ShapeDtypeStruct((_NG, _N, 128), jnp.float32),
        mesh=mesh,
        scratch_types=[
            pltpu.VMEM((_CHUNKS_PER_TILE, 128), jnp.int32),
            pltpu.VMEM((128, 128), jnp.float32),
            pltpu.VMEM((128, 128), jnp.float32),
            pltpu.SemaphoreType.DMA,
            pltpu.VMEM_SHARED((_N, 128), jnp.float32),
        ],
    )


def _scatter(msg, dstr, zeros):
    return _get_sc_scatter()(msg, dstr, zeros)


def kernel(positions, params):
    B, P, _ = positions.shape
    pos = positions.reshape(-1, 3)
    dst, phi = _knn_phi_call(pos, pos.T)
    dstr = dst.reshape(512, 128)
    zeros = jnp.zeros((_N, 128), jnp.float32)

    x = jnp.concatenate([
        jnp.broadcast_to(params["w_embed"], (_N, 32)),
        jnp.zeros((_N, _HID_OFF[-1] - 32), jnp.float32)], axis=1)

    layers = params["layers"]
    BTs = [_build_BT(lp) for lp in layers]
    mixws = [_build_mix_weights(lp) for lp in layers]
    bnw_e = [jnp.concatenate([
        lp["bn_w"][0:32],
        jnp.repeat(lp["bn_w"][32:48], 3),
        jnp.repeat(lp["bn_w"][48:56], 5)]).reshape(1, 120) for lp in layers]
    bnb = [lp["bn_b"].reshape(1, 32) for lp in layers]

    msg = _combine0_call(x, BTs[0], phi)
    for li in range(3):
        acc = _scatter(msg.reshape(_NG, _N * _KN, 128), dstr, zeros)
        y, stats = _mix_stats_call(acc, *mixws[li])
        if li < 2:
            x, msg = _bn_combine_call(y, x, stats, bnw_e[li], bnb[li],
                                      BTs[li + 1], phi)
        else:
            A2p = jnp.pad(params["A2"], ((0, 0), (0, 6))).astype(jnp.bfloat16)
            b2p = jnp.pad(params["b2"], (0, 6)).reshape(1, 8)
            out = _bn_head_call(y, x, stats, bnw_e[li], bnb[li],
                                params["Wf"].astype(jnp.bfloat16),
                                params["A1"].astype(jnp.bfloat16),
                                params["b1"].reshape(1, 16), A2p, b2p)
    return out[:, :2].reshape(B, P, 2)
